# baseline (speedup 1.0000x reference)
import functools

import jax
import jax.numpy as jnp
from jax import lax
from jax.experimental import pallas as pl
from jax.experimental.pallas import tpu as pltpu
from jax.experimental.pallas import tpu_sc as plsc

NUM_TOKENS = 20
EMBED_DIM = 64
LANES = 16
NUM_CORES = 2
NUM_SUBCORES = 16
NUM_WORKERS = NUM_CORES * NUM_SUBCORES


def _tree_sum(vals):
    while len(vals) > 1:
        nxt = [vals[i] + vals[i + 1] for i in range(0, len(vals) - 1, 2)]
        if len(vals) % 2:
            nxt.append(vals[-1])
        vals = nxt
    return vals[0]


def kernel(indices, assignment, token_emb):
    batch = indices.shape[0]
    vocab = token_emb.shape[0]
    b_per_w = batch // NUM_WORKERS
    n_groups = EMBED_DIM // LANES

    mesh = plsc.VectorSubcoreMesh(core_axis_name="c", subcore_axis_name="s")

    @functools.partial(
        pl.kernel,
        mesh=mesh,
        compiler_params=pltpu.CompilerParams(
            needs_layout_passes=False, use_tc_tiling_on_sc=False),
        out_type=jax.ShapeDtypeStruct((batch, EMBED_DIM), jnp.float32),
        scratch_types=[
            pltpu.VMEM((b_per_w,), jnp.int32),
            pltpu.VMEM((b_per_w, NUM_TOKENS), jnp.int32),
            pltpu.VMEM((vocab, EMBED_DIM), jnp.float32),
            pltpu.VMEM((b_per_w, EMBED_DIM), jnp.float32),
            pltpu.SemaphoreType.DMA,
            pltpu.SemaphoreType.DMA,
        ],
    )
    def nodepiece(idx_hbm, asg_hbm, emb_hbm, out_hbm,
                  idx_v, ids_v, emb_v, out_v, sem_emb, sem_ids):
        wid = lax.axis_index("s") * NUM_CORES + lax.axis_index("c")
        base = wid * b_per_w
        emb_cp = pltpu.async_copy(emb_hbm, emb_v, sem_emb)
        pltpu.sync_copy(idx_hbm.at[pl.ds(base, b_per_w)], idx_v)
        ids_cp = pltpu.async_copy(asg_hbm.at[idx_v], ids_v, sem_ids)
        ids_cp.wait()
        emb_cp.wait()

        inv = jnp.float32(1.0 / NUM_TOKENS)

        def ent_body(e, carry):
            va = ids_v[e, pl.ds(0, LANES)]
            vb = ids_v[e, pl.ds(NUM_TOKENS - LANES, LANES)]
            tids = ([va[j] for j in range(LANES)]
                    + [vb[j] for j in range(12, LANES)])
            for g in range(n_groups):
                vals = [emb_v[tids[t], pl.ds(g * LANES, LANES)]
                        for t in range(NUM_TOKENS)]
                out_v[e, pl.ds(g * LANES, LANES)] = _tree_sum(vals) * inv
            return carry

        lax.fori_loop(0, b_per_w, ent_body, 0)
        pltpu.sync_copy(out_v, out_hbm.at[pl.ds(base, b_per_w)])

    return nodepiece(indices, assignment, token_emb)


# speedup vs baseline: 1.9240x; 1.9240x over previous
import functools

import jax
import jax.numpy as jnp
from jax import lax
from jax.experimental import pallas as pl
from jax.experimental.pallas import tpu as pltpu
from jax.experimental.pallas import tpu_sc as plsc

NUM_TOKENS = 20
EMBED_DIM = 64
LANES = 16
NUM_CORES = 2
NUM_SUBCORES = 16
NUM_WORKERS = NUM_CORES * NUM_SUBCORES


def _tree_sum(vals):
    while len(vals) > 1:
        nxt = [vals[i] + vals[i + 1] for i in range(0, len(vals) - 1, 2)]
        if len(vals) % 2:
            nxt.append(vals[-1])
        vals = nxt
    return vals[0]


def kernel(indices, assignment, token_emb):
    batch = indices.shape[0]
    vocab = token_emb.shape[0]
    num_entities = assignment.shape[0]
    b_per_w = batch // NUM_WORKERS
    n_blocks = b_per_w // LANES
    n_groups = EMBED_DIM // LANES

    mesh = plsc.VectorSubcoreMesh(core_axis_name="c", subcore_axis_name="s")

    @functools.partial(
        pl.kernel,
        mesh=mesh,
        compiler_params=pltpu.CompilerParams(needs_layout_passes=False),
        out_type=jax.ShapeDtypeStruct((batch * EMBED_DIM,), jnp.float32),
        scratch_types=[
            pltpu.VMEM((b_per_w,), jnp.int32),
            pltpu.VMEM((b_per_w * NUM_TOKENS,), jnp.int32),
            pltpu.VMEM((b_per_w * NUM_TOKENS,), jnp.int32),
            pltpu.VMEM((vocab * EMBED_DIM,), jnp.float32),
            pltpu.VMEM((b_per_w * EMBED_DIM,), jnp.float32),
            pltpu.SemaphoreType.DMA,
            pltpu.SemaphoreType.DMA,
        ],
    )
    def nodepiece(idx_hbm, asg_hbm, emb_hbm, out_hbm,
                  idx_v, gidx_v, ids_v, emb_v, out_v, sem_emb, sem_ids):
        wid = lax.axis_index("s") * NUM_CORES + lax.axis_index("c")
        base = wid * b_per_w
        emb_cp = pltpu.async_copy(emb_hbm, emb_v, sem_emb)
        pltpu.sync_copy(idx_hbm.at[pl.ds(base, b_per_w)], idx_v)
        for blk in range(n_blocks):
            ev = idx_v[pl.ds(blk * LANES, LANES)]
            for t in range(NUM_TOKENS):
                gidx_v[pl.ds(t * b_per_w + blk * LANES, LANES)] = (
                    ev + t * num_entities)
        ids_cp = pltpu.async_copy(asg_hbm.at[gidx_v], ids_v, sem_ids)
        ids_cp.wait()
        emb_cp.wait()

        inv = jnp.float32(1.0 / NUM_TOKENS)

        def block_body(blk, carry):
            e0 = blk * LANES
            ob = blk * (LANES * EMBED_DIM)
            idvecs = [ids_v[pl.ds(t * b_per_w + e0, LANES)]
                      for t in range(NUM_TOKENS)]
            for j in range(LANES):
                tids = [idvecs[t][j] * EMBED_DIM for t in range(NUM_TOKENS)]
                for g in range(n_groups):
                    vals = [emb_v[pl.ds(tids[t] + g * LANES, LANES)]
                            for t in range(NUM_TOKENS)]
                    out_v[pl.ds(ob + j * EMBED_DIM + g * LANES, LANES)] = (
                        _tree_sum(vals) * inv)
            return carry

        lax.fori_loop(0, n_blocks, block_body, 0)
        pltpu.sync_copy(out_v, out_hbm.at[pl.ds(base * EMBED_DIM,
                                                b_per_w * EMBED_DIM)])

    out_flat = nodepiece(indices, assignment.T.reshape(-1),
                         token_emb.reshape(-1))
    return out_flat.reshape(batch, EMBED_DIM)


# speedup vs baseline: 2.0176x; 1.0486x over previous
import functools

import jax
import jax.numpy as jnp
from jax import lax
from jax.experimental import pallas as pl
from jax.experimental.pallas import tpu as pltpu
from jax.experimental.pallas import tpu_sc as plsc

NUM_TOKENS = 20
EMBED_DIM = 64
LANES = 16
NUM_CORES = 2
NUM_SUBCORES = 16
NUM_WORKERS = NUM_CORES * NUM_SUBCORES


def _tree_sum(vals):
    while len(vals) > 1:
        nxt = [vals[i] + vals[i + 1] for i in range(0, len(vals) - 1, 2)]
        if len(vals) % 2:
            nxt.append(vals[-1])
        vals = nxt
    return vals[0]


def kernel(indices, assignment, token_emb):
    batch = indices.shape[0]
    vocab = token_emb.shape[0]
    num_entities = assignment.shape[0]
    b_per_w = batch // NUM_WORKERS
    n_blocks = b_per_w // LANES
    n_groups = EMBED_DIM // LANES
    blk_sz = LANES * NUM_TOKENS

    mesh = plsc.VectorSubcoreMesh(core_axis_name="c", subcore_axis_name="s")

    @functools.partial(
        pl.kernel,
        mesh=mesh,
        compiler_params=pltpu.CompilerParams(needs_layout_passes=False),
        out_type=jax.ShapeDtypeStruct((batch * EMBED_DIM,), jnp.float32),
        scratch_types=[
            pltpu.VMEM((b_per_w,), jnp.int32),
            pltpu.VMEM((b_per_w * NUM_TOKENS,), jnp.int32),
            pltpu.VMEM((b_per_w * NUM_TOKENS,), jnp.int32),
            pltpu.VMEM_SHARED((vocab, EMBED_DIM), jnp.float32),
            pltpu.VMEM((blk_sz, EMBED_DIM), jnp.float32),
            pltpu.VMEM((b_per_w * EMBED_DIM,), jnp.float32),
            pltpu.SemaphoreType.DMA,
            pltpu.SemaphoreType.DMA,
            pltpu.SemaphoreType.DMA,
        ],
    )
    def nodepiece(idx_hbm, asg_hbm, emb_hbm, out_hbm,
                  idx_v, gidx_v, ids_v, emb_v, rows_v, out_v,
                  sem_emb, sem_ids, sem_rows):
        sid = lax.axis_index("s")
        wid = sid * NUM_CORES + lax.axis_index("c")
        base = wid * b_per_w
        rows_per_tile = (vocab // NUM_SUBCORES) // 8 * 8
        tail = vocab - rows_per_tile * NUM_SUBCORES

        @pl.when(sid < NUM_SUBCORES - 1)
        def _():
            pltpu.async_copy(
                emb_hbm.at[pl.ds(sid * rows_per_tile, rows_per_tile)],
                emb_v.at[pl.ds(sid * rows_per_tile, rows_per_tile)],
                sem_emb).wait()

        @pl.when(sid == NUM_SUBCORES - 1)
        def _():
            pltpu.async_copy(
                emb_hbm.at[pl.ds((NUM_SUBCORES - 1) * rows_per_tile,
                                 rows_per_tile + tail)],
                emb_v.at[pl.ds((NUM_SUBCORES - 1) * rows_per_tile,
                               rows_per_tile + tail)],
                sem_emb).wait()

        pltpu.sync_copy(idx_hbm.at[pl.ds(base, b_per_w)], idx_v)
        for blk in range(n_blocks):
            ev = idx_v[pl.ds(blk * LANES, LANES)]
            for t in range(NUM_TOKENS):
                gidx_v[pl.ds(blk * blk_sz + t * LANES, LANES)] = (
                    ev + t * num_entities)
        ids_cp = pltpu.async_copy(asg_hbm.at[gidx_v], ids_v, sem_ids)
        ids_cp.wait()
        plsc.subcore_barrier()

        inv = jnp.float32(1.0 / NUM_TOKENS)

        def block_body(blk, carry):
            ob = blk * (LANES * EMBED_DIM)
            pltpu.async_copy(emb_v.at[ids_v.at[pl.ds(blk * blk_sz, blk_sz)]],
                             rows_v, sem_rows).wait()
            for j in range(LANES):
                for g in range(n_groups):
                    vals = [rows_v[t * LANES + j, pl.ds(g * LANES, LANES)]
                            for t in range(NUM_TOKENS)]
                    out_v[pl.ds(ob + j * EMBED_DIM + g * LANES, LANES)] = (
                        _tree_sum(vals) * inv)
            return carry

        lax.fori_loop(0, n_blocks, block_body, 0)
        pltpu.sync_copy(out_v, out_hbm.at[pl.ds(base * EMBED_DIM,
                                                b_per_w * EMBED_DIM)])

    out_flat = nodepiece(indices, assignment.T.reshape(-1), token_emb)
    return out_flat.reshape(batch, EMBED_DIM)
